# baseline (device time: 44266 ns/iter reference)
import jax
import jax.numpy as jnp
from jax import lax
from jax.experimental import pallas as pl
from jax.experimental.pallas import tpu as pltpu

N_DEV = 4
B, SQ, SKV = 2, 512, 512
HQ_LOC, DH = 8, 64
DHP = 128
DM = 768
DQ_LOC = HQ_LOC * DH
DQP = HQ_LOC * DHP
ROWS = B * SQ
CHUNK = ROWS // N_DEV


def kernel(x, Wq, K_ext, V_ext, Wo):
    i = lax.axis_index("i")
    f16 = jnp.bfloat16

    Wq_loc = lax.dynamic_slice(Wq, (0, i * DQ_LOC), (DM, DQ_LOC)) * 0.125
    Wq_pad = jnp.pad(Wq_loc.reshape(DM, HQ_LOC, DH).astype(f16),
                     ((0, 0), (0, 0), (0, DHP - DH))).reshape(DM, DQP)
    Wo_loc = lax.dynamic_slice(Wo, (i * DQ_LOC, 0), (DQ_LOC, DM))
    Wo_pad = jnp.pad(Wo_loc.reshape(HQ_LOC, DH, DM).astype(f16),
                     ((0, 0), (0, DHP - DH), (0, 0))).reshape(DQP, DM)
    x16 = x.astype(f16)
    Kt_pad = jnp.pad(K_ext.transpose(0, 2, 3, 1).astype(f16),
                     ((0, 0), (0, 0), (0, DHP - DH), (0, 0)))
    Vt = V_ext.transpose(0, 2, 1, 3).astype(f16)
    Vaug = jnp.concatenate(
        [Vt, jnp.ones((B, HQ_LOC, SKV, 1), f16),
         jnp.zeros((B, HQ_LOC, SKV, DHP - DH - 1), f16)], axis=-1)

    def body(x_ref, wq_ref, kt_ref, va_ref, wo_ref, out_ref,
             acc_ref, ctx_ref, snd_rs, rs_buf, ag_ref,
             send_sems, recv_sems):
        my = lax.axis_index("i")

        barrier_sem = pltpu.get_barrier_semaphore()
        for d in range(1, N_DEV):
            pl.semaphore_signal(
                barrier_sem, inc=1,
                device_id=(lax.rem(my + d, N_DEV),),
                device_id_type=pl.DeviceIdType.MESH,
            )
        pl.semaphore_wait(barrier_sem, N_DEV - 1)

        qi = lax.broadcasted_iota(jnp.int32, (SQ, SKV), 0)
        ki = lax.broadcasted_iota(jnp.int32, (SQ, SKV), 1)
        dd = qi - ki
        mask = ((dd <= 128) & (dd >= -128)) | (ki < 32) | (qi < 32)
        bias = jnp.where(mask, 0.0, -1e9).astype(jnp.float32)

        def rs_send(d):
            return pltpu.make_async_remote_copy(
                src_ref=snd_rs.at[d - 1],
                dst_ref=rs_buf.at[d - 1],
                send_sem=send_sems.at[d - 1],
                recv_sem=recv_sems.at[d - 1],
                device_id=(lax.rem(my + d, N_DEV),),
                device_id_type=pl.DeviceIdType.MESH,
            )

        def rs_stage_and_send(d):
            c = lax.rem(my + d, N_DEV)
            snd_rs[d - 1, :, :] = acc_ref[pl.ds(c * CHUNK, CHUNK),
                                          :].astype(f16)
            rs_send(d).start()

        def attn_rows(b, r0, nr):
            q = qs[b]
            for h in range(HQ_LOC):
                s = lax.dot_general(
                    q[r0:r0 + nr, h * DHP:(h + 1) * DHP], kt_ref[b, h, :, :],
                    (((1,), (0,)), ((), ())),
                    preferred_element_type=jnp.float32)
                w = jnp.exp(s + bias[r0:r0 + nr, :]).astype(f16)
                cd = jnp.dot(w, va_ref[b, h, :, :],
                             preferred_element_type=jnp.float32)
                ctx_ref[r0:r0 + nr, h * DHP:(h + 1) * DHP] = (
                    cd / cd[:, DH:DH + 1]).astype(f16)
            acc_ref[pl.ds(b * SQ + r0, nr), :] = jnp.dot(
                ctx_ref[r0:r0 + nr, :], wo_ref[:, :],
                preferred_element_type=jnp.float32)

        def send_ready(pred):
            for d in range(1, N_DEV):
                c = lax.rem(my + d, N_DEV)

                @pl.when(pred(c))
                def _(d=d):
                    rs_stage_and_send(d)

        qs = [jnp.dot(x_ref[b, :, :], wq_ref[:, :],
                      preferred_element_type=jnp.float32).astype(f16)
              for b in range(B)]

        attn_rows(0, 0, SQ)
        send_ready(lambda c: c < 2)
        attn_rows(1, 0, CHUNK)
        send_ready(lambda c: c == 2)
        attn_rows(1, CHUNK, CHUNK)
        send_ready(lambda c: c == 3)

        for d in range(1, N_DEV):
            rs_send(d).wait_recv()
        red = acc_ref[pl.ds(my * CHUNK, CHUNK), :]
        for d in range(1, N_DEV):
            red = red + rs_buf[d - 1, :, :].astype(jnp.float32)
        ag_ref[pl.ds(my * CHUNK, CHUNK), :] = red.astype(f16)

        ag_rdmas = []
        for d in range(1, N_DEV):
            r = pltpu.make_async_remote_copy(
                src_ref=ag_ref.at[pl.ds(my * CHUNK, CHUNK), :],
                dst_ref=ag_ref.at[pl.ds(my * CHUNK, CHUNK), :],
                send_sem=send_sems.at[N_DEV - 1 + d - 1],
                recv_sem=recv_sems.at[N_DEV - 1 + d - 1],
                device_id=(lax.rem(my + d, N_DEV),),
                device_id_type=pl.DeviceIdType.MESH,
            )
            r.start()
            ag_rdmas.append(r)

        for d in range(1, N_DEV):
            src = lax.rem(my - d + N_DEV, N_DEV)
            pltpu.make_async_remote_copy(
                src_ref=ag_ref.at[pl.ds(src * CHUNK, CHUNK), :],
                dst_ref=ag_ref.at[pl.ds(src * CHUNK, CHUNK), :],
                send_sem=send_sems.at[N_DEV - 1 + d - 1],
                recv_sem=recv_sems.at[N_DEV - 1 + d - 1],
                device_id=(src,),
                device_id_type=pl.DeviceIdType.MESH,
            ).wait_recv()

        for d in range(1, N_DEV):
            rs_send(d).wait_send()
        for r in ag_rdmas:
            r.wait_send()

        out_ref[0, :, :] = ag_ref[pl.ds(0, SQ), :].astype(jnp.float32)
        out_ref[1, :, :] = ag_ref[pl.ds(SQ, SQ), :].astype(jnp.float32)

    return pl.pallas_call(
        body,
        out_shape=jax.ShapeDtypeStruct((B, SQ, DM), jnp.float32),
        in_specs=[pl.BlockSpec(memory_space=pltpu.VMEM)] * 5,
        out_specs=pl.BlockSpec(memory_space=pltpu.VMEM),
        scratch_shapes=[
            pltpu.VMEM((ROWS, DM), jnp.float32),
            pltpu.VMEM((SQ, DQP), jnp.bfloat16),
            pltpu.VMEM((N_DEV - 1, CHUNK, DM), jnp.bfloat16),
            pltpu.VMEM((N_DEV - 1, CHUNK, DM), jnp.bfloat16),
            pltpu.VMEM((ROWS, DM), jnp.bfloat16),
            pltpu.SemaphoreType.DMA((2 * (N_DEV - 1),)),
            pltpu.SemaphoreType.DMA((2 * (N_DEV - 1),)),
        ],
        compiler_params=pltpu.CompilerParams(collective_id=0),
    )(x16, Wq_pad, Kt_pad, Vaug, Wo_pad)


# device time: 43768 ns/iter; 1.0114x vs baseline; 1.0114x over previous
import jax
import jax.numpy as jnp
from jax import lax
from jax.experimental import pallas as pl
from jax.experimental.pallas import tpu as pltpu

N_DEV = 4
B, SQ, SKV = 2, 512, 512
HQ_LOC, DH = 8, 64
DHP = 128
DM = 768
DQ_LOC = HQ_LOC * DH
DQP = HQ_LOC * DHP
ROWS = B * SQ
CHUNK = ROWS // N_DEV


def kernel(x, Wq, K_ext, V_ext, Wo):
    i = lax.axis_index("i")
    f16 = jnp.bfloat16

    Wq_loc = lax.dynamic_slice(Wq, (0, i * DQ_LOC), (DM, DQ_LOC)) * 0.125
    Wq_pad = jnp.pad(Wq_loc.reshape(DM, HQ_LOC, DH).astype(f16),
                     ((0, 0), (0, 0), (0, DHP - DH))).reshape(DM, DQP)
    Wo_loc = lax.dynamic_slice(Wo, (i * DQ_LOC, 0), (DQ_LOC, DM))
    Wo_pad = jnp.pad(Wo_loc.reshape(HQ_LOC, DH, DM).astype(f16),
                     ((0, 0), (0, DHP - DH), (0, 0))).reshape(DQP, DM)
    x16 = x.astype(f16)
    Kt_pad = jnp.pad(K_ext.transpose(0, 2, 3, 1).astype(f16),
                     ((0, 0), (0, 0), (0, DHP - DH), (0, 0)))
    Vt = V_ext.transpose(0, 2, 1, 3).astype(f16)
    Vaug = jnp.concatenate(
        [Vt, jnp.ones((B, HQ_LOC, SKV, 1), f16),
         jnp.zeros((B, HQ_LOC, SKV, DHP - DH - 1), f16)], axis=-1)

    def body(x_ref, wq_ref, kt_ref, va_ref, wo_ref, out_ref,
             acc_ref, ctx_ref, snd_rs, rs_buf, ag_ref,
             send_sems, recv_sems):
        my = lax.axis_index("i")

        barrier_sem = pltpu.get_barrier_semaphore()
        for d in range(1, N_DEV):
            pl.semaphore_signal(
                barrier_sem, inc=1,
                device_id=(lax.rem(my + d, N_DEV),),
                device_id_type=pl.DeviceIdType.MESH,
            )
        pl.semaphore_wait(barrier_sem, N_DEV - 1)

        qi = lax.broadcasted_iota(jnp.int32, (SQ, SKV), 0)
        ki = lax.broadcasted_iota(jnp.int32, (SQ, SKV), 1)
        dd = qi - ki
        mask = ((dd <= 128) & (dd >= -128)) | (ki < 32) | (qi < 32)
        bias = jnp.where(mask, 0.0, -1e9).astype(jnp.float32)

        def rs_send(d):
            return pltpu.make_async_remote_copy(
                src_ref=snd_rs.at[d - 1],
                dst_ref=rs_buf.at[d - 1],
                send_sem=send_sems.at[d - 1],
                recv_sem=recv_sems.at[d - 1],
                device_id=(lax.rem(my + d, N_DEV),),
                device_id_type=pl.DeviceIdType.MESH,
            )

        def rs_stage_and_send(d):
            c = lax.rem(my + d, N_DEV)
            snd_rs[d - 1, :, :] = acc_ref[pl.ds(c * CHUNK, CHUNK),
                                          :].astype(f16)
            rs_send(d).start()

        for b in range(B):
            q = jnp.dot(x_ref[b, :, :], wq_ref[:, :],
                        preferred_element_type=jnp.float32
                        ).astype(f16)
            for h in range(HQ_LOC):
                s = lax.dot_general(
                    q[:, h * DHP:(h + 1) * DHP], kt_ref[b, h, :, :],
                    (((1,), (0,)), ((), ())),
                    preferred_element_type=jnp.float32)
                w = jnp.exp(s + bias).astype(f16)
                cd = jnp.dot(w, va_ref[b, h, :, :],
                             preferred_element_type=jnp.float32)
                ctx_ref[:, h * DHP:(h + 1) * DHP] = (
                    cd / cd[:, DH:DH + 1]).astype(f16)
            acc_ref[pl.ds(b * SQ, SQ), :] = jnp.dot(
                ctx_ref[:, :], wo_ref[:, :],
                preferred_element_type=jnp.float32)

            for d in range(1, N_DEV):
                c = lax.rem(my + d, N_DEV)
                if b == 0:
                    @pl.when(c < 2)
                    def _(d=d):
                        rs_stage_and_send(d)
                else:
                    @pl.when(c >= 2)
                    def _(d=d):
                        rs_stage_and_send(d)

        for d in range(1, N_DEV):
            rs_send(d).wait_recv()
        red = acc_ref[pl.ds(my * CHUNK, CHUNK), :]
        for d in range(1, N_DEV):
            red = red + rs_buf[d - 1, :, :].astype(jnp.float32)
        ag_ref[pl.ds(my * CHUNK, CHUNK), :] = red.astype(f16)

        ag_rdmas = []
        for d in range(1, N_DEV):
            r = pltpu.make_async_remote_copy(
                src_ref=ag_ref.at[pl.ds(my * CHUNK, CHUNK), :],
                dst_ref=ag_ref.at[pl.ds(my * CHUNK, CHUNK), :],
                send_sem=send_sems.at[N_DEV - 1 + d - 1],
                recv_sem=recv_sems.at[N_DEV - 1 + d - 1],
                device_id=(lax.rem(my + d, N_DEV),),
                device_id_type=pl.DeviceIdType.MESH,
            )
            r.start()
            ag_rdmas.append(r)

        for d in range(1, N_DEV):
            src = lax.rem(my - d + N_DEV, N_DEV)
            pltpu.make_async_remote_copy(
                src_ref=ag_ref.at[pl.ds(src * CHUNK, CHUNK), :],
                dst_ref=ag_ref.at[pl.ds(src * CHUNK, CHUNK), :],
                send_sem=send_sems.at[N_DEV - 1 + d - 1],
                recv_sem=recv_sems.at[N_DEV - 1 + d - 1],
                device_id=(src,),
                device_id_type=pl.DeviceIdType.MESH,
            ).wait_recv()

        for d in range(1, N_DEV):
            rs_send(d).wait_send()
        for r in ag_rdmas:
            r.wait_send()

        out_ref[0, :, :] = ag_ref[pl.ds(0, SQ), :].astype(jnp.float32)
        out_ref[1, :, :] = ag_ref[pl.ds(SQ, SQ), :].astype(jnp.float32)

    return pl.pallas_call(
        body,
        out_shape=jax.ShapeDtypeStruct((B, SQ, DM), jnp.float32),
        in_specs=[pl.BlockSpec(memory_space=pltpu.VMEM)] * 5,
        out_specs=pl.BlockSpec(memory_space=pltpu.VMEM),
        scratch_shapes=[
            pltpu.VMEM((ROWS, DM), jnp.float32),
            pltpu.VMEM((SQ, DQP), jnp.bfloat16),
            pltpu.VMEM((N_DEV - 1, CHUNK, DM), jnp.bfloat16),
            pltpu.VMEM((N_DEV - 1, CHUNK, DM), jnp.bfloat16),
            pltpu.VMEM((ROWS, DM), jnp.bfloat16),
            pltpu.SemaphoreType.DMA((2 * (N_DEV - 1),)),
            pltpu.SemaphoreType.DMA((2 * (N_DEV - 1),)),
        ],
        compiler_params=pltpu.CompilerParams(collective_id=0),
    )(x16, Wq_pad, Kt_pad, Vaug, Wo_pad)
